# Initial kernel scaffold; baseline (speedup 1.0000x reference)
import jax
import jax.numpy as jnp
from jax.experimental import pallas as pl


def kernel(user_emb, item_emb, edge_index, edge_weight):
    raise NotImplementedError("write your pallas kernel here")


# speedup vs baseline: 2.8349x; 2.8349x over previous
import functools

import jax
import jax.numpy as jnp
from jax import lax
from jax.experimental import pallas as pl
from jax.experimental.pallas import tpu as pltpu
from jax.experimental.pallas import tpu_sc as plsc

U = 25000
HALF = 25088
ROWS_PER_TEC = HALF // 16
NPAD = 2 * HALF
GAP = HALF - U
DIM = 64
E = 800000
CHUNK = 128
CHUNKS_PER_TEC = -(-E // (16 * CHUNK))
EPAD = CHUNKS_PER_TEC * CHUNK * 16
DUMMY = U + 8


@functools.partial(
    pl.kernel,
    out_type=jax.ShapeDtypeStruct((NPAD, DIM), jnp.float32),
    mesh=plsc.VectorSubcoreMesh(core_axis_name="c", subcore_axis_name="s"),
    compiler_params=pltpu.CompilerParams(use_tc_tiling_on_sc=False),
    scratch_types=[
        pltpu.VMEM((CHUNK,), jnp.int32),
        pltpu.VMEM((CHUNK,), jnp.int32),
        pltpu.VMEM((CHUNK,), jnp.float32),
        pltpu.VMEM((CHUNK, DIM), jnp.float32),
        pltpu.VMEM_SHARED((HALF, DIM), jnp.float32),
        pltpu.SemaphoreType.DMA,
    ],
)
def _propagate(table, src, dst, w, out, srcv, dstv, wv, rowsv, acc, sem):
    c = lax.axis_index("c")
    s = lax.axis_index("s")
    lo = c * U

    zero16 = jnp.zeros((16,), jnp.float32)

    def _zero_rowsv(r, carry):
        for b in range(4):
            rowsv[r, pl.ds(b * 16, 16)] = zero16
        return carry

    lax.fori_loop(0, CHUNK, _zero_rowsv, 0)

    abase = s * ROWS_PER_TEC
    for k in range(12):
        pltpu.sync_copy(rowsv, acc.at[pl.ds(abase + k * CHUNK, CHUNK)])
    pltpu.sync_copy(rowsv.at[pl.ds(0, 32)], acc.at[pl.ds(abase + 12 * CHUNK, 32)])
    plsc.subcore_barrier()

    def _chunk(i, carry):
        ebase = (s * CHUNKS_PER_TEC + i) * CHUNK
        pltpu.sync_copy(src.at[pl.ds(ebase, CHUNK)], srcv)
        pltpu.sync_copy(dst.at[pl.ds(ebase, CHUNK)], dstv)
        pltpu.sync_copy(w.at[pl.ds(ebase, CHUNK)], wv)
        for j in range(8):
            sv = srcv[pl.ds(j * 16, 16)]
            srcv[pl.ds(j * 16, 16)] = jnp.where(sv >= U, sv + GAP, sv)
            dvec = dstv[pl.ds(j * 16, 16)] - lo
            inr = (dvec >= 0) & (dvec < U)
            dstv[pl.ds(j * 16, 16)] = jnp.where(inr, dvec, DUMMY)
        pltpu.async_copy(table.at[srcv], rowsv, sem).wait()
        for j in range(8):
            wvec = wv[pl.ds(j * 16, 16)]
            for t in range(16):
                ws = jnp.broadcast_to(wvec[t], (16,))
                r = j * 16 + t
                for b in range(4):
                    rowsv[r, pl.ds(b * 16, 16)] = rowsv[r, pl.ds(b * 16, 16)] * ws
        pltpu.sync_copy(rowsv, acc.at[dstv], add=True)
        return carry

    lax.fori_loop(0, CHUNKS_PER_TEC, _chunk, 0)
    plsc.subcore_barrier()

    obase = c * HALF + abase
    for k in range(12):
        pltpu.sync_copy(acc.at[pl.ds(abase + k * CHUNK, CHUNK)],
                        out.at[pl.ds(obase + k * CHUNK, CHUNK)])
    pltpu.sync_copy(acc.at[pl.ds(abase + 12 * CHUNK, 32)],
                    out.at[pl.ds(obase + 12 * CHUNK, 32)])


def kernel(user_emb, item_emb, edge_index, edge_weight):
    src = edge_index[0].astype(jnp.int32)
    dst = edge_index[1].astype(jnp.int32)
    w = edge_weight.astype(jnp.float32)
    pad = EPAD - E
    src = jnp.concatenate([src, jnp.zeros((pad,), jnp.int32)])
    dst = jnp.concatenate([dst, jnp.zeros((pad,), jnp.int32)])
    w = jnp.concatenate([w, jnp.zeros((pad,), jnp.float32)])
    gap = jnp.zeros((GAP, DIM), jnp.float32)
    e0 = jnp.concatenate([user_emb, gap, item_emb, gap], axis=0)
    e1 = _propagate(e0, src, dst, w)
    e2 = _propagate(e1, src, dst, w)
    light = (e0 + e1 + e2) * (1.0 / 3.0)
    return light[:U], light[HALF:HALF + U]


# speedup vs baseline: 4.8982x; 1.7279x over previous
import functools

import jax
import jax.numpy as jnp
from jax import lax
from jax.experimental import pallas as pl
from jax.experimental.pallas import tpu as pltpu
from jax.experimental.pallas import tpu_sc as plsc

U = 25000
HALF = 25088
ROWS_PER_TEC = HALF // 16
NPAD = 2 * HALF
GAP = HALF - U
DIM = 64
E = 800000
SUB = 128
SB = 14
SBS_PER_TEC = 28
SUBS_PER_TEC = SB * SBS_PER_TEC
EPAD = SUBS_PER_TEC * SUB * 16
EROWS = EPAD // SUB
DUMMY = U + 8


@functools.partial(
    pl.kernel,
    out_type=jax.ShapeDtypeStruct((NPAD, DIM), jnp.float32),
    mesh=plsc.VectorSubcoreMesh(core_axis_name="c", subcore_axis_name="s"),
    compiler_params=pltpu.CompilerParams(use_tc_tiling_on_sc=False),
    scratch_types=[
        pltpu.VMEM((SB, SUB), jnp.int32),
        pltpu.VMEM((SB, SUB), jnp.int32),
        pltpu.VMEM((SB, SUB), jnp.float32),
        pltpu.VMEM((2, SUB), jnp.int32),
        pltpu.VMEM((2, SUB), jnp.int32),
        pltpu.VMEM((2, SUB), jnp.float32),
        pltpu.VMEM((2, SUB, DIM), jnp.float32),
        pltpu.VMEM_SHARED((HALF, DIM), jnp.float32),
        pltpu.SemaphoreType.DMA,
        pltpu.SemaphoreType.DMA,
    ],
)
def _propagate(table, src, dst, w, out, esrc, edst, ew, srcadj, dstloc,
               wring, rowsv, acc, gsem, ssem):
    c = lax.axis_index("c")
    s = lax.axis_index("s")
    lo = c * U

    zero16 = jnp.zeros((16,), jnp.float32)

    def _zero_rowsv(r, carry):
        for b in range(4):
            rowsv[0, r, pl.ds(b * 16, 16)] = zero16
        return carry

    lax.fori_loop(0, SUB, _zero_rowsv, 0)

    abase = s * ROWS_PER_TEC
    for k in range(12):
        pltpu.sync_copy(rowsv.at[0], acc.at[pl.ds(abase + k * SUB, SUB)])
    pltpu.sync_copy(rowsv.at[0].at[pl.ds(0, 32)],
                    acc.at[pl.ds(abase + 12 * SUB, 32)])
    plsc.subcore_barrier()

    row0 = s * SUBS_PER_TEC

    def _load_sb(sb):
        base = row0 + sb * SB
        pltpu.sync_copy(src.at[pl.ds(base, SB)], esrc)
        pltpu.sync_copy(dst.at[pl.ds(base, SB)], edst)
        pltpu.sync_copy(w.at[pl.ds(base, SB)], ew)

    def _prep(n):
        kk = lax.rem(n, SB)
        p = lax.rem(n, 2)
        for g in range(8):
            sl = pl.ds(g * 16, 16)
            sv = esrc[kk, sl]
            srcadj[p, sl] = jnp.where(sv >= U, sv + GAP, sv)
            dv = edst[kk, sl] - lo
            inr = (dv >= 0) & (dv < U)
            dstloc[p, sl] = jnp.where(inr, dv, DUMMY)
            wring[p, sl] = ew[kk, sl]

    def _fire_gather(p):
        pltpu.async_copy(table.at[srcadj.at[p]], rowsv.at[p], gsem)

    def _wait_gather(p):
        pltpu.make_async_copy(table.at[srcadj.at[p]], rowsv.at[p],
                              gsem).wait()

    def _fire_scatter(p):
        pltpu.async_copy(rowsv.at[p], acc.at[dstloc.at[p]], ssem, add=True)

    def _wait_scatter(p):
        pltpu.make_async_copy(rowsv.at[p], acc.at[dstloc.at[p]],
                              ssem).wait()

    def _scale(p):
        def body(g, carry):
            wvec = wring[p, pl.ds(g * 16, 16)]
            for e in range(16):
                ws = jnp.broadcast_to(wvec[e], (16,))
                r = g * 16 + e
                for b in range(4):
                    rowsv[p, r, pl.ds(b * 16, 16)] = (
                        rowsv[p, r, pl.ds(b * 16, 16)] * ws)
            return carry
        lax.fori_loop(0, 8, body, 0)

    _load_sb(0)
    _prep(0)
    _fire_gather(0)

    def _step(i, carry):
        nxt = i + 1
        p = lax.rem(i, 2)
        pn = lax.rem(nxt, 2)

        @pl.when(nxt < SUBS_PER_TEC)
        def _():
            @pl.when(i >= 1)
            def _():
                _wait_scatter(pn)

            @pl.when(lax.rem(nxt, SB) == 0)
            def _():
                _load_sb(lax.div(nxt, SB))

            _prep(nxt)
            _fire_gather(pn)

        _wait_gather(p)
        _scale(p)
        _fire_scatter(p)
        return carry

    lax.fori_loop(0, SUBS_PER_TEC, _step, 0)
    _wait_scatter(0)
    _wait_scatter(1)
    plsc.subcore_barrier()

    obase = c * HALF + abase
    for k in range(12):
        pltpu.sync_copy(acc.at[pl.ds(abase + k * SUB, SUB)],
                        out.at[pl.ds(obase + k * SUB, SUB)])
    pltpu.sync_copy(acc.at[pl.ds(abase + 12 * SUB, 32)],
                    out.at[pl.ds(obase + 12 * SUB, 32)])


def kernel(user_emb, item_emb, edge_index, edge_weight):
    src = edge_index[0].astype(jnp.int32)
    dst = edge_index[1].astype(jnp.int32)
    w = edge_weight.astype(jnp.float32)
    pad = EPAD - E
    src = jnp.concatenate([src, jnp.zeros((pad,), jnp.int32)]).reshape(EROWS, SUB)
    dst = jnp.concatenate([dst, jnp.zeros((pad,), jnp.int32)]).reshape(EROWS, SUB)
    w = jnp.concatenate([w, jnp.zeros((pad,), jnp.float32)]).reshape(EROWS, SUB)
    gap = jnp.zeros((GAP, DIM), jnp.float32)
    e0 = jnp.concatenate([user_emb, gap, item_emb, gap], axis=0)
    e1 = _propagate(e0, src, dst, w)
    e2 = _propagate(e1, src, dst, w)
    light = (e0 + e1 + e2) * (1.0 / 3.0)
    return light[:U], light[HALF:HALF + U]
